# baseline (speedup 1.0000x reference)
import functools

import jax
import jax.numpy as jnp
from jax import lax
from jax.experimental import pallas as pl
from jax.experimental.pallas import tpu as pltpu
from jax.experimental.pallas import tpu_sc as plsc

K = 8
H = 128
W = 128
P = 4096
C = 64
NPIX = 2 * H * W
NSLOT = NPIX * K
PTAB = P + 512
WSZ = 256

_CHUNK = 256

_GDN = jax.lax.GatherDimensionNumbers(
    offset_dims=(), collapsed_slice_dims=(0,), start_index_map=(0,))


def _lane_bcast(vec, i):
    idx = jnp.full((16, 1), i, jnp.int32)
    return jax.lax.gather(
        vec, idx, _GDN, (1,),
        mode=jax.lax.GatherScatterMode.PROMISE_IN_BOUNDS)


def _sort_kernel(pts_ref, par_ref, srt_ref, lo_ref, hi_ref):
    hf = par_ref[2:3, 0:1]
    arrs = [pts_ref[0, c] for c in range(4)]
    py = arrs[1]
    v = (hf * (1.0 - py) - 1.0) * 0.5

    yl3 = jax.lax.broadcasted_iota(jnp.int32, (1, 1, 128), 2).astype(jnp.float32)
    v3 = v[:, :, None]
    lo = jnp.sum((v3 < yl3 - 1.5).astype(jnp.int32), axis=(0, 1)).reshape(1, 128)
    hi = jnp.sum((v3 < yl3 + 1.5).astype(jnp.int32), axis=(0, 1)).reshape(1, 128)
    lo_ref[0] = lo
    hi_ref[0] = hi

    fi = (jax.lax.broadcasted_iota(jnp.int32, (32, 128), 0) * 128
          + jax.lax.broadcasted_iota(jnp.int32, (32, 128), 1))
    key = v
    for k in [2 ** e for e in range(1, 13)]:
        kl = k.bit_length() - 1
        for j in [k // 2 >> s for s in range(0, 20) if (k // 2 >> s) >= 1]:
            jl = j.bit_length() - 1
            is_lo = (fi & j) == 0
            take_min = (((fi >> jl) ^ (fi >> kl)) & 1) == 0
            if j >= 128:
                m = j // 128
                def flip(a, m=m):
                    a4 = a.reshape(32 // (2 * m), 2, m, 128)
                    a4 = jnp.concatenate([a4[:, 1:2], a4[:, 0:1]], axis=1)
                    return a4.reshape(32, 128)
                pkey = flip(key)
                parrs = [flip(a) for a in arrs]
            else:
                def rollp(a, j=j):
                    lbit = is_lo
                    return jnp.where(lbit, pltpu.roll(a, 128 - j, 1),
                                     pltpu.roll(a, j, 1))
                pkey = rollp(key)
                parrs = [rollp(a) for a in arrs]
            swap = ((take_min & (pkey < key))
                    | (jnp.logical_not(take_min) & (pkey > key)))
            key = jnp.where(take_min, jnp.minimum(key, pkey),
                            jnp.maximum(key, pkey))
            arrs = [jnp.where(swap, pa, a) for pa, a in zip(parrs, arrs)]

    for c in range(4):
        srt_ref[0, c] = arrs[c]


def _raster_kernel(tab_ref, xs_ref, ys_ref, lo_ref, hi_ref, w_ref, i_ref):
    b = pl.program_id(0)
    y = pl.program_id(1)
    xs = xs_ref[0:1, :]
    r2 = xs_ref[1:2, 0:1]
    ysc = ys_ref[0][0:1, 0:1]
    lane = jax.lax.broadcasted_iota(jnp.int32, (1, W), 1)

    lo = lo_ref[b, 0, y]
    hi = hi_ref[b, 0, y]
    lo8 = (lo // 8) * 8
    nw = jnp.where(hi > lo, (hi - lo8 + WSZ - 1) // WSZ, 0)

    inf = jnp.float32(jnp.inf)
    z8 = jnp.full((K, W), inf, jnp.float32)
    d8 = jnp.zeros((K, W), jnp.float32)
    o8 = jnp.full((K, W), 1.0e9, jnp.float32)

    def wbody(wi, state):
        z8, d8, o8 = state
        start = pl.multiple_of(lo8 + wi * WSZ, 8)
        px = tab_ref[0, pl.ds(start, WSZ), 0:1]
        py = tab_ref[0, pl.ds(start, WSZ), 1:2]
        pz = tab_ref[0, pl.ds(start, WSZ), 2:3]
        oid = tab_ref[0, pl.ds(start, WSZ), 3:4]
        dx = xs - px
        dy = ysc - py
        d2 = dx * dx + dy * dy
        valid = (d2 < r2) & (pz > 0.0)
        zc = jnp.concatenate([z8, jnp.where(valid, pz, inf)], axis=0)
        dc = jnp.concatenate([d8, d2], axis=0)
        oc = jnp.concatenate([o8, jnp.where(valid, oid, 1.0e9)], axis=0)

        zrows, drows, orows = [], [], []
        for _ in range(K):
            zmin = jnp.min(zc, axis=0, keepdims=True)
            eq = zc == zmin
            omin = jnp.min(jnp.where(eq, oc, 1.0e9), axis=0, keepdims=True)
            sel = eq & (oc == omin)
            dmin = jnp.min(jnp.where(sel, dc, inf), axis=0, keepdims=True)
            dmin = jnp.where(zmin < inf, dmin, 0.0)
            zrows.append(zmin)
            drows.append(dmin)
            orows.append(omin)
            zc = jnp.where(sel, inf, zc)
        return (jnp.concatenate(zrows, axis=0),
                jnp.concatenate(drows, axis=0),
                jnp.concatenate(orows, axis=0))

    z8, d8, o8 = lax.fori_loop(0, nw, wbody, (z8, d8, o8))

    ok = z8 < inf
    dist = jnp.where(ok, d8, -1.0) / r2
    a = 1.0 - jnp.sqrt(jnp.clip(dist, 0.001, 1.0))
    a = jnp.where(ok, a, 0.0)
    wrows = []
    trans = jnp.ones((1, W), jnp.float32)
    for k in range(K):
        ak = a[k:k + 1, :]
        wrows.append(ak * trans)
        trans = trans * (1.0 - ak)
    krow = jax.lax.broadcasted_iota(jnp.int32, (K, W), 0)
    dummy = (lane * 31 + y * 7 + krow * 523) & (P - 1)
    oidi = jnp.where(ok, o8, 0.0).astype(jnp.int32)
    gidx = jnp.where(ok, oidi, dummy) + b * P
    w_ref[0, 0] = jnp.concatenate(wrows, axis=0)
    i_ref[0, 0] = gidx


def _rasterize(pts3D, image_size):
    Hf = jnp.asarray(image_size[0], dtype=jnp.float32)
    Wf = jnp.asarray(image_size[1], dtype=jnp.float32)
    bs = pts3D.shape[0]
    radius = 1.5 / Hf * 2.0
    r2 = radius * radius

    pts = pts3D * jnp.array([-1.0, -1.0, 1.0], dtype=pts3D.dtype)
    oid = jnp.broadcast_to(
        jnp.arange(P, dtype=jnp.float32)[None, :, None], (bs, P, 1))
    pts_sq = jnp.transpose(
        jnp.concatenate([pts, oid], axis=2), (0, 2, 1)).reshape(bs, 4, 32, 128)

    xs = 1.0 - (2.0 * jnp.arange(W, dtype=jnp.float32) + 1.0) / Wf
    ys = 1.0 - (2.0 * jnp.arange(H, dtype=jnp.float32) + 1.0) / Hf
    par = jnp.zeros((8, 128), dtype=jnp.float32)
    par = par.at[0, :].set(xs)
    par = par.at[1, :].set(r2)
    par = par.at[2, :].set(Hf)
    ys_arr = jnp.broadcast_to(ys[:, None, None], (H, 1, 128)).astype(jnp.float32)

    srt, lo, hi = pl.pallas_call(
        _sort_kernel,
        grid=(bs,),
        in_specs=[
            pl.BlockSpec((1, 4, 32, 128), lambda b: (b, 0, 0, 0)),
            pl.BlockSpec((8, 128), lambda b: (0, 0)),
        ],
        out_specs=[
            pl.BlockSpec((1, 4, 32, 128), lambda b: (b, 0, 0, 0)),
            pl.BlockSpec((1, 1, 128), lambda b: (b, 0, 0)),
            pl.BlockSpec((1, 1, 128), lambda b: (b, 0, 0)),
        ],
        out_shape=[
            jax.ShapeDtypeStruct((bs, 4, 32, 128), jnp.float32),
            jax.ShapeDtypeStruct((bs, 1, 128), jnp.int32),
            jax.ShapeDtypeStruct((bs, 1, 128), jnp.int32),
        ],
    )(pts_sq, par)

    st = jnp.transpose(srt.reshape(bs, 4, P), (0, 2, 1))
    tab = jnp.zeros((bs, PTAB, 128), jnp.float32)
    tab = tab.at[:, :P, 0:4].set(st)

    wgt, gidx = pl.pallas_call(
        _raster_kernel,
        grid=(bs, H),
        in_specs=[
            pl.BlockSpec((1, PTAB, 128), lambda b, y: (b, 0, 0)),
            pl.BlockSpec((8, 128), lambda b, y: (0, 0)),
            pl.BlockSpec((1, 1, 128), lambda b, y: (y, 0, 0)),
            pl.BlockSpec(memory_space=pltpu.SMEM),
            pl.BlockSpec(memory_space=pltpu.SMEM),
        ],
        out_specs=[
            pl.BlockSpec((1, 1, K, W), lambda b, y: (b, y, 0, 0)),
            pl.BlockSpec((1, 1, K, W), lambda b, y: (b, y, 0, 0)),
        ],
        out_shape=[
            jax.ShapeDtypeStruct((bs, H, K, W), jnp.float32),
            jax.ShapeDtypeStruct((bs, H, K, W), jnp.int32),
        ],
    )(tab, par, ys_arr, lo, hi)
    nz = jnp.zeros((272,), jnp.int32)
    nz = nz.at[:bs * H].set((hi > lo).astype(jnp.int32).reshape(-1))
    return wgt, gidx, nz


def _make_sc_composite():
    info = plsc.get_sparse_core_info()
    nc, ns = info.num_cores, info.num_subcores
    nw = nc * ns
    slots_per_w = NSLOT // nw
    nchunk = slots_per_w // _CHUNK
    mesh = plsc.VectorSubcoreMesh(core_axis_name="c", subcore_axis_name="s")

    @functools.partial(
        pl.kernel,
        mesh=mesh,
        out_type=jax.ShapeDtypeStruct((NPIX, C), jnp.float32),
        scratch_types=[
            pltpu.VMEM((_CHUNK,), jnp.int32),
            pltpu.VMEM((_CHUNK,), jnp.float32),
            pltpu.VMEM((_CHUNK, 2 * C), jnp.float32),
            pltpu.VMEM((_CHUNK // K, C), jnp.float32),
            pltpu.SemaphoreType.DMA,
        ],
    )
    def composite(idx_hbm, w_hbm, feats_hbm, nz_hbm, out_hbm, idx_v, w_v,
                  rows_v, acc_v, sem):
        del nz_hbm
        wid = lax.axis_index("s") * nc + lax.axis_index("c")
        base = wid * slots_per_w

        def chunk_body(ci, _):
            off = pl.multiple_of(base + ci * _CHUNK, _CHUNK)
            pltpu.sync_copy(idx_hbm.at[pl.ds(off, _CHUNK)], idx_v)
            pltpu.sync_copy(w_hbm.at[pl.ds(off, _CHUNK)], w_v)
            pltpu.async_copy(feats_hbm.at[idx_v], rows_v, sem).wait()
            for pair in range(_CHUNK // 16):
                r0 = pair * 16
                wblk = w_v[pl.ds(r0, 16)]
                for sub in range(2):
                    px = pair * 2 + sub
                    for cs in range(C // 16):
                        acc = jnp.zeros((16,), jnp.float32)
                        for k in range(K):
                            wv = _lane_bcast(wblk, sub * K + k)
                            acc = acc + wv * rows_v[
                                r0 + sub * K + k, pl.ds(cs * 16, 16)]
                        acc_v[px, pl.ds(cs * 16, 16)] = acc
            pltpu.sync_copy(
                acc_v,
                out_hbm.at[pl.ds(pl.multiple_of(off // K, _CHUNK // K),
                                 _CHUNK // K)])
            return ()

        lax.fori_loop(0, nchunk, chunk_body, ())

    return composite


def kernel(pts3D, src, image_size):
    bs = pts3D.shape[0]
    wgt, gidx, nz = _rasterize(pts3D, image_size)
    feats = jnp.zeros((bs * P, 2 * C), jnp.float32)
    feats = feats.at[:, :C].set(jnp.transpose(src, (0, 2, 1)).reshape(bs * P, C))
    idx_flat = jnp.transpose(gidx, (0, 1, 3, 2)).reshape(NSLOT)
    w_flat = jnp.transpose(wgt, (0, 1, 3, 2)).reshape(NSLOT)
    out = _make_sc_composite()(idx_flat, w_flat, feats, nz)
    out = out.reshape(bs, H, W, C)
    return jnp.transpose(out, (0, 3, 1, 2))


# speedup vs baseline: 1.0013x; 1.0013x over previous
import functools

import jax
import jax.numpy as jnp
from jax import lax
from jax.experimental import pallas as pl
from jax.experimental.pallas import tpu as pltpu
from jax.experimental.pallas import tpu_sc as plsc

K = 8
H = 128
W = 128
P = 4096
C = 64
NPIX = 2 * H * W
NSLOT = NPIX * K
PTAB = P + 512
WSZ = 256

_CHUNK = 256

_GDN = jax.lax.GatherDimensionNumbers(
    offset_dims=(), collapsed_slice_dims=(0,), start_index_map=(0,))


def _lane_bcast(vec, i):
    idx = jnp.full((16, 1), i, jnp.int32)
    return jax.lax.gather(
        vec, idx, _GDN, (1,),
        mode=jax.lax.GatherScatterMode.PROMISE_IN_BOUNDS)


def _sort_kernel(pts_ref, par_ref, srt_ref, lo_ref, hi_ref):
    hf = par_ref[2:3, 0:1]
    arrs = [pts_ref[0, c] for c in range(4)]
    py = arrs[1]
    v = (hf * (1.0 - py) - 1.0) * 0.5

    yl3 = jax.lax.broadcasted_iota(jnp.int32, (1, 1, 128), 2).astype(jnp.float32)
    v3 = v[:, :, None]
    lo = jnp.sum((v3 < yl3 - 1.5).astype(jnp.int32), axis=(0, 1)).reshape(1, 128)
    hi = jnp.sum((v3 < yl3 + 1.5).astype(jnp.int32), axis=(0, 1)).reshape(1, 128)
    lo_ref[0] = lo
    hi_ref[0] = hi

    fi = (jax.lax.broadcasted_iota(jnp.int32, (32, 128), 0) * 128
          + jax.lax.broadcasted_iota(jnp.int32, (32, 128), 1))
    key = v
    for k in [2 ** e for e in range(1, 13)]:
        kl = k.bit_length() - 1
        for j in [k // 2 >> s for s in range(0, 20) if (k // 2 >> s) >= 1]:
            jl = j.bit_length() - 1
            is_lo = (fi & j) == 0
            take_min = (((fi >> jl) ^ (fi >> kl)) & 1) == 0
            if j >= 128:
                m = j // 128
                def flip(a, m=m):
                    a4 = a.reshape(32 // (2 * m), 2, m, 128)
                    a4 = jnp.concatenate([a4[:, 1:2], a4[:, 0:1]], axis=1)
                    return a4.reshape(32, 128)
                pkey = flip(key)
                parrs = [flip(a) for a in arrs]
            else:
                def rollp(a, j=j):
                    lbit = is_lo
                    return jnp.where(lbit, pltpu.roll(a, 128 - j, 1),
                                     pltpu.roll(a, j, 1))
                pkey = rollp(key)
                parrs = [rollp(a) for a in arrs]
            swap = ((take_min & (pkey < key))
                    | (jnp.logical_not(take_min) & (pkey > key)))
            key = jnp.where(take_min, jnp.minimum(key, pkey),
                            jnp.maximum(key, pkey))
            arrs = [jnp.where(swap, pa, a) for pa, a in zip(parrs, arrs)]

    for c in range(4):
        srt_ref[0, c] = arrs[c]


def _raster_kernel(tab_ref, xs_ref, ys_ref, lo_ref, hi_ref, w_ref, i_ref):
    b = pl.program_id(0)
    y = pl.program_id(1)
    xs = xs_ref[0:1, :]
    r2 = xs_ref[1:2, 0:1]
    ysc = ys_ref[0][0:1, 0:1]
    lane = jax.lax.broadcasted_iota(jnp.int32, (1, W), 1)

    lo = lo_ref[b, 0, y]
    hi = hi_ref[b, 0, y]
    lo8 = (lo // 8) * 8
    nw = jnp.where(hi > lo, (hi - lo8 + WSZ - 1) // WSZ, 0)

    inf = jnp.float32(jnp.inf)
    z8 = jnp.full((K, W), inf, jnp.float32)
    d8 = jnp.zeros((K, W), jnp.float32)
    o8 = jnp.full((K, W), 1.0e9, jnp.float32)

    def wbody(wi, state):
        z8, d8, o8 = state
        start = pl.multiple_of(lo8 + wi * WSZ, 8)
        px = tab_ref[0, pl.ds(start, WSZ), 0:1]
        py = tab_ref[0, pl.ds(start, WSZ), 1:2]
        pz = tab_ref[0, pl.ds(start, WSZ), 2:3]
        oid = tab_ref[0, pl.ds(start, WSZ), 3:4]
        dx = xs - px
        dy = ysc - py
        d2 = dx * dx + dy * dy
        valid = (d2 < r2) & (pz > 0.0)
        zc = jnp.concatenate([z8, jnp.where(valid, pz, inf)], axis=0)
        dc = jnp.concatenate([d8, d2], axis=0)
        oc = jnp.concatenate([o8, jnp.where(valid, oid, 1.0e9)], axis=0)

        zrows, drows, orows = [], [], []
        for _ in range(K):
            zmin = jnp.min(zc, axis=0, keepdims=True)
            eq = zc == zmin
            omin = jnp.min(jnp.where(eq, oc, 1.0e9), axis=0, keepdims=True)
            sel = eq & (oc == omin)
            dmin = jnp.min(jnp.where(sel, dc, inf), axis=0, keepdims=True)
            dmin = jnp.where(zmin < inf, dmin, 0.0)
            zrows.append(zmin)
            drows.append(dmin)
            orows.append(omin)
            zc = jnp.where(sel, inf, zc)
        return (jnp.concatenate(zrows, axis=0),
                jnp.concatenate(drows, axis=0),
                jnp.concatenate(orows, axis=0))

    z8, d8, o8 = lax.fori_loop(0, nw, wbody, (z8, d8, o8))

    ok = z8 < inf
    dist = jnp.where(ok, d8, -1.0) / r2
    a = 1.0 - jnp.sqrt(jnp.clip(dist, 0.001, 1.0))
    a = jnp.where(ok, a, 0.0)
    wrows = []
    trans = jnp.ones((1, W), jnp.float32)
    for k in range(K):
        ak = a[k:k + 1, :]
        wrows.append(ak * trans)
        trans = trans * (1.0 - ak)
    krow = jax.lax.broadcasted_iota(jnp.int32, (K, W), 0)
    dummy = (lane * 31 + y * 7 + krow * 523) & (P - 1)
    oidi = jnp.where(ok, o8, 0.0).astype(jnp.int32)
    gidx = jnp.where(ok, oidi, dummy) + b * P
    w_ref[0, 0] = jnp.concatenate(wrows, axis=0)
    i_ref[0, 0] = gidx


def _rasterize(pts3D, image_size):
    Hf = jnp.asarray(image_size[0], dtype=jnp.float32)
    Wf = jnp.asarray(image_size[1], dtype=jnp.float32)
    bs = pts3D.shape[0]
    radius = 1.5 / Hf * 2.0
    r2 = radius * radius

    pts = pts3D * jnp.array([-1.0, -1.0, 1.0], dtype=pts3D.dtype)
    oid = jnp.broadcast_to(
        jnp.arange(P, dtype=jnp.float32)[None, :, None], (bs, P, 1))
    pts_sq = jnp.transpose(
        jnp.concatenate([pts, oid], axis=2), (0, 2, 1)).reshape(bs, 4, 32, 128)

    xs = 1.0 - (2.0 * jnp.arange(W, dtype=jnp.float32) + 1.0) / Wf
    ys = 1.0 - (2.0 * jnp.arange(H, dtype=jnp.float32) + 1.0) / Hf
    par = jnp.zeros((8, 128), dtype=jnp.float32)
    par = par.at[0, :].set(xs)
    par = par.at[1, :].set(r2)
    par = par.at[2, :].set(Hf)
    ys_arr = jnp.broadcast_to(ys[:, None, None], (H, 1, 128)).astype(jnp.float32)

    srt, lo, hi = pl.pallas_call(
        _sort_kernel,
        grid=(bs,),
        in_specs=[
            pl.BlockSpec((1, 4, 32, 128), lambda b: (b, 0, 0, 0)),
            pl.BlockSpec((8, 128), lambda b: (0, 0)),
        ],
        out_specs=[
            pl.BlockSpec((1, 4, 32, 128), lambda b: (b, 0, 0, 0)),
            pl.BlockSpec((1, 1, 128), lambda b: (b, 0, 0)),
            pl.BlockSpec((1, 1, 128), lambda b: (b, 0, 0)),
        ],
        out_shape=[
            jax.ShapeDtypeStruct((bs, 4, 32, 128), jnp.float32),
            jax.ShapeDtypeStruct((bs, 1, 128), jnp.int32),
            jax.ShapeDtypeStruct((bs, 1, 128), jnp.int32),
        ],
    )(pts_sq, par)

    st = jnp.transpose(srt.reshape(bs, 4, P), (0, 2, 1))
    tab = jnp.zeros((bs, PTAB, 128), jnp.float32)
    tab = tab.at[:, :P, 0:4].set(st)

    wgt, gidx = pl.pallas_call(
        _raster_kernel,
        grid=(bs, H),
        in_specs=[
            pl.BlockSpec((1, PTAB, 128), lambda b, y: (b, 0, 0)),
            pl.BlockSpec((8, 128), lambda b, y: (0, 0)),
            pl.BlockSpec((1, 1, 128), lambda b, y: (y, 0, 0)),
            pl.BlockSpec(memory_space=pltpu.SMEM),
            pl.BlockSpec(memory_space=pltpu.SMEM),
        ],
        out_specs=[
            pl.BlockSpec((1, 1, K, W), lambda b, y: (b, y, 0, 0)),
            pl.BlockSpec((1, 1, K, W), lambda b, y: (b, y, 0, 0)),
        ],
        out_shape=[
            jax.ShapeDtypeStruct((bs, H, K, W), jnp.float32),
            jax.ShapeDtypeStruct((bs, H, K, W), jnp.int32),
        ],
    )(tab, par, ys_arr, lo, hi)
    nz = jnp.zeros((272,), jnp.int32)
    nz = nz.at[:bs * H].set((hi > lo).astype(jnp.int32).reshape(-1))
    return wgt, gidx, nz


def _make_sc_composite():
    info = plsc.get_sparse_core_info()
    nc, ns = info.num_cores, info.num_subcores
    nw = nc * ns
    slots_per_w = NSLOT // nw
    nchunk = slots_per_w // _CHUNK
    mesh = plsc.VectorSubcoreMesh(core_axis_name="c", subcore_axis_name="s")

    @functools.partial(
        pl.kernel,
        mesh=mesh,
        out_type=jax.ShapeDtypeStruct((NPIX, C), jnp.float32),
        scratch_types=[
            pltpu.VMEM((_CHUNK,), jnp.int32),
            pltpu.VMEM((_CHUNK,), jnp.int32),
            pltpu.VMEM((_CHUNK,), jnp.float32),
            pltpu.VMEM((_CHUNK,), jnp.float32),
            pltpu.VMEM((_CHUNK, 2 * C), jnp.float32),
            pltpu.VMEM((_CHUNK, 2 * C), jnp.float32),
            pltpu.VMEM((_CHUNK // K, C), jnp.float32),
            pltpu.SemaphoreType.DMA,
            pltpu.SemaphoreType.DMA,
        ],
    )
    def composite(idx_hbm, w_hbm, feats_hbm, nz_hbm, out_hbm, idx_a, idx_b,
                  w_a, w_b, rows_a, rows_b, acc_v, sem_a, sem_b):
        del nz_hbm
        wid = lax.axis_index("s") * nc + lax.axis_index("c")
        base = wid * slots_per_w

        def compute(rows_v, w_v, off):
            for pair in range(_CHUNK // 16):
                r0 = pair * 16
                wblk = w_v[pl.ds(r0, 16)]
                for sub in range(2):
                    px = pair * 2 + sub
                    for cs in range(C // 16):
                        acc = jnp.zeros((16,), jnp.float32)
                        for k in range(K):
                            wv = _lane_bcast(wblk, sub * K + k)
                            acc = acc + wv * rows_v[
                                r0 + sub * K + k, pl.ds(cs * 16, 16)]
                        acc_v[px, pl.ds(cs * 16, 16)] = acc
            pltpu.sync_copy(
                acc_v,
                out_hbm.at[pl.ds(pl.multiple_of(off // K, _CHUNK // K),
                                 _CHUNK // K)])

        def load(buf_i, buf_w, off):
            pltpu.sync_copy(idx_hbm.at[pl.ds(off, _CHUNK)], buf_i)
            pltpu.sync_copy(w_hbm.at[pl.ds(off, _CHUNK)], buf_w)

        npair = nchunk // 2
        off0 = pl.multiple_of(base, _CHUNK)
        load(idx_a, w_a, off0)
        gather_a = pltpu.async_copy(feats_hbm.at[idx_a], rows_a, sem_a)

        def pair_body(g, _):
            off_e = pl.multiple_of(base + (2 * g) * _CHUNK, _CHUNK)
            off_o = pl.multiple_of(off_e + _CHUNK, _CHUNK)
            off_n = pl.multiple_of(off_o + _CHUNK, _CHUNK)
            load(idx_b, w_b, off_o)
            cp_b = pltpu.async_copy(feats_hbm.at[idx_b], rows_b, sem_b)
            pltpu.make_async_copy(feats_hbm.at[idx_a], rows_a, sem_a).wait()
            compute(rows_a, w_a, off_e)

            @pl.when(g < npair - 1)
            def _():
                load(idx_a, w_a, off_n)
                pltpu.async_copy(feats_hbm.at[idx_a], rows_a, sem_a)

            cp_b.wait()
            compute(rows_b, w_b, off_o)
            return ()

        lax.fori_loop(0, npair, pair_body, ())
        del gather_a

    return composite


def kernel(pts3D, src, image_size):
    bs = pts3D.shape[0]
    wgt, gidx, nz = _rasterize(pts3D, image_size)
    feats = jnp.zeros((bs * P, 2 * C), jnp.float32)
    feats = feats.at[:, :C].set(jnp.transpose(src, (0, 2, 1)).reshape(bs * P, C))
    idx_flat = jnp.transpose(gidx, (0, 1, 3, 2)).reshape(NSLOT)
    w_flat = jnp.transpose(wgt, (0, 1, 3, 2)).reshape(NSLOT)
    out = _make_sc_composite()(idx_flat, w_flat, feats, nz)
    out = out.reshape(bs, H, W, C)
    return jnp.transpose(out, (0, 3, 1, 2))


# speedup vs baseline: 1.1640x; 1.1625x over previous
import functools

import jax
import jax.numpy as jnp
from jax import lax
from jax.experimental import pallas as pl
from jax.experimental.pallas import tpu as pltpu
from jax.experimental.pallas import tpu_sc as plsc

K = 8
H = 128
W = 128
P = 4096
C = 64
NPIX = 2 * H * W
NSLOT = NPIX * K
PTAB = P + 512
WSZ = 256

_CHUNK = 256

_GDN = jax.lax.GatherDimensionNumbers(
    offset_dims=(), collapsed_slice_dims=(0,), start_index_map=(0,))


def _lane_bcast(vec, i):
    idx = jnp.full((16, 1), i, jnp.int32)
    return jax.lax.gather(
        vec, idx, _GDN, (1,),
        mode=jax.lax.GatherScatterMode.PROMISE_IN_BOUNDS)


def _sort_kernel(pts_ref, par_ref, srt_ref, lo_ref, hi_ref):
    hf = par_ref[2:3, 0:1]
    arrs = [pts_ref[0, c] for c in range(4)]
    py = arrs[1]
    v = (hf * (1.0 - py) - 1.0) * 0.5

    yl3 = jax.lax.broadcasted_iota(jnp.int32, (1, 1, 128), 2).astype(jnp.float32)
    v3 = v[:, :, None]
    lo = jnp.sum((v3 < yl3 - 1.5).astype(jnp.int32), axis=(0, 1)).reshape(1, 128)
    hi = jnp.sum((v3 < yl3 + 1.5).astype(jnp.int32), axis=(0, 1)).reshape(1, 128)
    lo_ref[0] = lo
    hi_ref[0] = hi

    fi = (jax.lax.broadcasted_iota(jnp.int32, (32, 128), 0) * 128
          + jax.lax.broadcasted_iota(jnp.int32, (32, 128), 1))
    key = v
    for k in [2 ** e for e in range(1, 13)]:
        kl = k.bit_length() - 1
        for j in [k // 2 >> s for s in range(0, 20) if (k // 2 >> s) >= 1]:
            jl = j.bit_length() - 1
            is_lo = (fi & j) == 0
            take_min = (((fi >> jl) ^ (fi >> kl)) & 1) == 0
            if j >= 128:
                m = j // 128
                def flip(a, m=m):
                    a4 = a.reshape(32 // (2 * m), 2, m, 128)
                    a4 = jnp.concatenate([a4[:, 1:2], a4[:, 0:1]], axis=1)
                    return a4.reshape(32, 128)
                pkey = flip(key)
                parrs = [flip(a) for a in arrs]
            else:
                def rollp(a, j=j):
                    lbit = is_lo
                    return jnp.where(lbit, pltpu.roll(a, 128 - j, 1),
                                     pltpu.roll(a, j, 1))
                pkey = rollp(key)
                parrs = [rollp(a) for a in arrs]
            swap = ((take_min & (pkey < key))
                    | (jnp.logical_not(take_min) & (pkey > key)))
            key = jnp.where(take_min, jnp.minimum(key, pkey),
                            jnp.maximum(key, pkey))
            arrs = [jnp.where(swap, pa, a) for pa, a in zip(parrs, arrs)]

    for c in range(4):
        srt_ref[0, c] = arrs[c]


def _raster_kernel(tab_ref, xs_ref, ys_ref, lo_ref, hi_ref, w_ref, i_ref):
    y = pl.program_id(0)
    xs = xs_ref[0:1, :]
    r2 = xs_ref[1:2, 0:1]
    boff = xs_ref[3:4, 0:1]
    ysc = ys_ref[0][0:1, 0:1]
    lane = jax.lax.broadcasted_iota(jnp.int32, (1, W), 1)

    lo = lo_ref[0, 0, y]
    hi = hi_ref[0, 0, y]
    lo8 = (lo // 8) * 8
    nw = jnp.where(hi > lo, (hi - lo8 + WSZ - 1) // WSZ, 0)

    inf = jnp.float32(jnp.inf)
    z8 = jnp.full((K, W), inf, jnp.float32)
    d8 = jnp.zeros((K, W), jnp.float32)
    o8 = jnp.full((K, W), 1.0e9, jnp.float32)

    def wbody(wi, state):
        z8, d8, o8 = state
        start = pl.multiple_of(lo8 + wi * WSZ, 8)
        px = tab_ref[0, pl.ds(start, WSZ), 0:1]
        py = tab_ref[0, pl.ds(start, WSZ), 1:2]
        pz = tab_ref[0, pl.ds(start, WSZ), 2:3]
        oid = tab_ref[0, pl.ds(start, WSZ), 3:4]
        dx = xs - px
        dy = ysc - py
        d2 = dx * dx + dy * dy
        valid = (d2 < r2) & (pz > 0.0)
        zc = jnp.concatenate([z8, jnp.where(valid, pz, inf)], axis=0)
        dc = jnp.concatenate([d8, d2], axis=0)
        oc = jnp.concatenate([o8, jnp.where(valid, oid, 1.0e9)], axis=0)

        zrows, drows, orows = [], [], []
        for _ in range(K):
            zmin = jnp.min(zc, axis=0, keepdims=True)
            eq = zc == zmin
            omin = jnp.min(jnp.where(eq, oc, 1.0e9), axis=0, keepdims=True)
            sel = eq & (oc == omin)
            dmin = jnp.min(jnp.where(sel, dc, inf), axis=0, keepdims=True)
            dmin = jnp.where(zmin < inf, dmin, 0.0)
            zrows.append(zmin)
            drows.append(dmin)
            orows.append(omin)
            zc = jnp.where(sel, inf, zc)
        return (jnp.concatenate(zrows, axis=0),
                jnp.concatenate(drows, axis=0),
                jnp.concatenate(orows, axis=0))

    z8, d8, o8 = lax.fori_loop(0, nw, wbody, (z8, d8, o8))

    ok = z8 < inf
    dist = jnp.where(ok, d8, -1.0) / r2
    a = 1.0 - jnp.sqrt(jnp.clip(dist, 0.001, 1.0))
    a = jnp.where(ok, a, 0.0)
    wrows = []
    trans = jnp.ones((1, W), jnp.float32)
    for k in range(K):
        ak = a[k:k + 1, :]
        wrows.append(ak * trans)
        trans = trans * (1.0 - ak)
    krow = jax.lax.broadcasted_iota(jnp.int32, (K, W), 0)
    dummy = (lane * 31 + y * 7 + krow * 523) & (P - 1)
    oidi = jnp.where(ok, o8, 0.0).astype(jnp.int32)
    gidx = jnp.where(ok, oidi, dummy) + boff.astype(jnp.int32)
    w_ref[0, 0] = jnp.concatenate(wrows, axis=0)
    i_ref[0, 0] = gidx


def _rasterize(pts3D, image_size):
    Hf = jnp.asarray(image_size[0], dtype=jnp.float32)
    Wf = jnp.asarray(image_size[1], dtype=jnp.float32)
    bs = pts3D.shape[0]
    radius = 1.5 / Hf * 2.0
    r2 = radius * radius

    pts = pts3D * jnp.array([-1.0, -1.0, 1.0], dtype=pts3D.dtype)
    oid = jnp.broadcast_to(
        jnp.arange(P, dtype=jnp.float32)[None, :, None], (bs, P, 1))
    pts_sq = jnp.transpose(
        jnp.concatenate([pts, oid], axis=2), (0, 2, 1)).reshape(bs, 4, 32, 128)

    xs = 1.0 - (2.0 * jnp.arange(W, dtype=jnp.float32) + 1.0) / Wf
    ys = 1.0 - (2.0 * jnp.arange(H, dtype=jnp.float32) + 1.0) / Hf
    par = jnp.zeros((8, 128), dtype=jnp.float32)
    par = par.at[0, :].set(xs)
    par = par.at[1, :].set(r2)
    par = par.at[2, :].set(Hf)
    pars = [par.at[3, :].set(float(b * P)) for b in range(bs)]
    ys_arr = jnp.broadcast_to(ys[:, None, None], (H, 1, 128)).astype(jnp.float32)

    srt, lo, hi = pl.pallas_call(
        _sort_kernel,
        grid=(bs,),
        in_specs=[
            pl.BlockSpec((1, 4, 32, 128), lambda b: (b, 0, 0, 0)),
            pl.BlockSpec((8, 128), lambda b: (0, 0)),
        ],
        out_specs=[
            pl.BlockSpec((1, 4, 32, 128), lambda b: (b, 0, 0, 0)),
            pl.BlockSpec((1, 1, 128), lambda b: (b, 0, 0)),
            pl.BlockSpec((1, 1, 128), lambda b: (b, 0, 0)),
        ],
        out_shape=[
            jax.ShapeDtypeStruct((bs, 4, 32, 128), jnp.float32),
            jax.ShapeDtypeStruct((bs, 1, 128), jnp.int32),
            jax.ShapeDtypeStruct((bs, 1, 128), jnp.int32),
        ],
    )(pts_sq, par)

    st = jnp.transpose(srt.reshape(bs, 4, P), (0, 2, 1))
    tab = jnp.zeros((bs, PTAB, 128), jnp.float32)
    tab = tab.at[:, :P, 0:4].set(st)

    outs = []
    for b in range(bs):
        wgt_b, gidx_b = pl.pallas_call(
            _raster_kernel,
            grid=(H,),
            in_specs=[
                pl.BlockSpec((1, PTAB, 128), lambda y: (0, 0, 0)),
                pl.BlockSpec((8, 128), lambda y: (0, 0)),
                pl.BlockSpec((1, 1, 128), lambda y: (y, 0, 0)),
                pl.BlockSpec(memory_space=pltpu.SMEM),
                pl.BlockSpec(memory_space=pltpu.SMEM),
            ],
            out_specs=[
                pl.BlockSpec((1, 1, K, W), lambda y: (0, y, 0, 0)),
                pl.BlockSpec((1, 1, K, W), lambda y: (0, y, 0, 0)),
            ],
            out_shape=[
                jax.ShapeDtypeStruct((1, H, K, W), jnp.float32),
                jax.ShapeDtypeStruct((1, H, K, W), jnp.int32),
            ],
        )(tab[b:b + 1], pars[b], ys_arr, lo[b:b + 1], hi[b:b + 1])
        outs.append((wgt_b, gidx_b))
    return outs


def _make_sc_composite(npix):
    nslot = npix * K
    info = plsc.get_sparse_core_info()
    nc, ns = info.num_cores, info.num_subcores
    nw = nc * ns
    slots_per_w = nslot // nw
    nchunk = slots_per_w // _CHUNK
    mesh = plsc.VectorSubcoreMesh(core_axis_name="c", subcore_axis_name="s")

    @functools.partial(
        pl.kernel,
        mesh=mesh,
        out_type=jax.ShapeDtypeStruct((npix, C), jnp.float32),
        scratch_types=[
            pltpu.VMEM((_CHUNK,), jnp.int32),
            pltpu.VMEM((_CHUNK,), jnp.int32),
            pltpu.VMEM((_CHUNK,), jnp.float32),
            pltpu.VMEM((_CHUNK,), jnp.float32),
            pltpu.VMEM((_CHUNK, 2 * C), jnp.float32),
            pltpu.VMEM((_CHUNK, 2 * C), jnp.float32),
            pltpu.VMEM((_CHUNK // K, C), jnp.float32),
            pltpu.SemaphoreType.DMA,
            pltpu.SemaphoreType.DMA,
        ],
    )
    def composite(idx_hbm, w_hbm, feats_hbm, out_hbm, idx_a, idx_b,
                  w_a, w_b, rows_a, rows_b, acc_v, sem_a, sem_b):
        wid = lax.axis_index("s") * nc + lax.axis_index("c")
        base = wid * slots_per_w

        def compute(rows_v, w_v, off):
            for pair in range(_CHUNK // 16):
                r0 = pair * 16
                wblk = w_v[pl.ds(r0, 16)]
                for sub in range(2):
                    px = pair * 2 + sub
                    for cs in range(C // 16):
                        acc = jnp.zeros((16,), jnp.float32)
                        for k in range(K):
                            wv = _lane_bcast(wblk, sub * K + k)
                            acc = acc + wv * rows_v[
                                r0 + sub * K + k, pl.ds(cs * 16, 16)]
                        acc_v[px, pl.ds(cs * 16, 16)] = acc
            pltpu.sync_copy(
                acc_v,
                out_hbm.at[pl.ds(pl.multiple_of(off // K, _CHUNK // K),
                                 _CHUNK // K)])

        def load(buf_i, buf_w, off):
            pltpu.sync_copy(idx_hbm.at[pl.ds(off, _CHUNK)], buf_i)
            pltpu.sync_copy(w_hbm.at[pl.ds(off, _CHUNK)], buf_w)

        npair = nchunk // 2
        off0 = pl.multiple_of(base, _CHUNK)
        load(idx_a, w_a, off0)
        gather_a = pltpu.async_copy(feats_hbm.at[idx_a], rows_a, sem_a)

        def pair_body(g, _):
            off_e = pl.multiple_of(base + (2 * g) * _CHUNK, _CHUNK)
            off_o = pl.multiple_of(off_e + _CHUNK, _CHUNK)
            off_n = pl.multiple_of(off_o + _CHUNK, _CHUNK)
            load(idx_b, w_b, off_o)
            cp_b = pltpu.async_copy(feats_hbm.at[idx_b], rows_b, sem_b)
            pltpu.make_async_copy(feats_hbm.at[idx_a], rows_a, sem_a).wait()
            compute(rows_a, w_a, off_e)

            @pl.when(g < npair - 1)
            def _():
                load(idx_a, w_a, off_n)
                pltpu.async_copy(feats_hbm.at[idx_a], rows_a, sem_a)

            cp_b.wait()
            compute(rows_b, w_b, off_o)
            return ()

        lax.fori_loop(0, npair, pair_body, ())
        del gather_a

    return composite


def kernel(pts3D, src, image_size):
    bs = pts3D.shape[0]
    per_batch = _rasterize(pts3D, image_size)
    feats = jnp.zeros((bs * P, 2 * C), jnp.float32)
    feats = feats.at[:, :C].set(jnp.transpose(src, (0, 2, 1)).reshape(bs * P, C))
    npix_b = H * W
    comp = _make_sc_composite(npix_b)
    outs = []
    for wgt_b, gidx_b in per_batch:
        idx_flat = jnp.transpose(gidx_b, (0, 1, 3, 2)).reshape(npix_b * K)
        w_flat = jnp.transpose(wgt_b, (0, 1, 3, 2)).reshape(npix_b * K)
        outs.append(comp(idx_flat, w_flat, feats))
    out = jnp.stack(outs).reshape(bs, H, W, C)
    return jnp.transpose(out, (0, 3, 1, 2))


# speedup vs baseline: 1.2801x; 1.0998x over previous
import functools

import jax
import jax.numpy as jnp
from jax import lax
from jax.experimental import pallas as pl
from jax.experimental.pallas import tpu as pltpu
from jax.experimental.pallas import tpu_sc as plsc

K = 8
H = 128
W = 128
P = 4096
C = 64
NPIX = 2 * H * W
NSLOT = NPIX * K
PTAB = P + 512
WSZ = 256

_CHUNK = 256

_GDN = jax.lax.GatherDimensionNumbers(
    offset_dims=(), collapsed_slice_dims=(0,), start_index_map=(0,))


def _lane_bcast(vec, i):
    idx = jnp.full((16, 1), i, jnp.int32)
    return jax.lax.gather(
        vec, idx, _GDN, (1,),
        mode=jax.lax.GatherScatterMode.PROMISE_IN_BOUNDS)


def _sort_kernel(pts_ref, par_ref, srt_ref, lo_ref, hi_ref):
    hf = par_ref[2:3, 0:1]
    arrs = [pts_ref[0, c] for c in range(4)]
    py = arrs[1]
    v = (hf * (1.0 - py) - 1.0) * 0.5

    yl3 = jax.lax.broadcasted_iota(jnp.int32, (1, 1, 128), 2).astype(jnp.float32)
    v3 = v[:, :, None]
    lo = jnp.sum((v3 < yl3 - 1.5).astype(jnp.int32), axis=(0, 1)).reshape(1, 128)
    hi = jnp.sum((v3 < yl3 + 1.5).astype(jnp.int32), axis=(0, 1)).reshape(1, 128)
    lo_ref[0] = lo
    hi_ref[0] = hi

    fi = (jax.lax.broadcasted_iota(jnp.int32, (32, 128), 0) * 128
          + jax.lax.broadcasted_iota(jnp.int32, (32, 128), 1))
    key = v
    for k in [2 ** e for e in range(1, 13)]:
        kl = k.bit_length() - 1
        for j in [k // 2 >> s for s in range(0, 20) if (k // 2 >> s) >= 1]:
            jl = j.bit_length() - 1
            is_lo = (fi & j) == 0
            take_min = (((fi >> jl) ^ (fi >> kl)) & 1) == 0
            if j >= 128:
                m = j // 128
                def flip(a, m=m):
                    a4 = a.reshape(32 // (2 * m), 2, m, 128)
                    a4 = jnp.concatenate([a4[:, 1:2], a4[:, 0:1]], axis=1)
                    return a4.reshape(32, 128)
                pkey = flip(key)
                parrs = [flip(a) for a in arrs]
            else:
                def rollp(a, j=j):
                    lbit = is_lo
                    return jnp.where(lbit, pltpu.roll(a, 128 - j, 1),
                                     pltpu.roll(a, j, 1))
                pkey = rollp(key)
                parrs = [rollp(a) for a in arrs]
            swap = ((take_min & (pkey < key))
                    | (jnp.logical_not(take_min) & (pkey > key)))
            key = jnp.where(take_min, jnp.minimum(key, pkey),
                            jnp.maximum(key, pkey))
            arrs = [jnp.where(swap, pa, a) for pa, a in zip(parrs, arrs)]

    for c in range(4):
        srt_ref[0, c] = arrs[c]


def _raster_kernel(tab_ref, xs_ref, ys_ref, lo_ref, hi_ref, w_ref, i_ref):
    y = pl.program_id(0)
    xs = xs_ref[0:1, :]
    r2 = xs_ref[1:2, 0:1]
    boff = xs_ref[3:4, 0:1]
    ysc = ys_ref[0][0:1, 0:1]
    lane = jax.lax.broadcasted_iota(jnp.int32, (1, W), 1)

    lo = lo_ref[0, 0, y]
    hi = hi_ref[0, 0, y]
    lo8 = (lo // 8) * 8
    nw = jnp.where(hi > lo, (hi - lo8 + WSZ - 1) // WSZ, 0)

    inf = jnp.float32(jnp.inf)
    z8 = jnp.full((K, W), inf, jnp.float32)
    d8 = jnp.zeros((K, W), jnp.float32)
    o8 = jnp.full((K, W), 1.0e9, jnp.float32)

    def wbody(wi, state):
        z8, d8, o8 = state
        start = pl.multiple_of(lo8 + wi * WSZ, 8)
        px = tab_ref[0, pl.ds(start, WSZ), 0:1]
        py = tab_ref[0, pl.ds(start, WSZ), 1:2]
        pz = tab_ref[0, pl.ds(start, WSZ), 2:3]
        oid = tab_ref[0, pl.ds(start, WSZ), 3:4]
        dx = xs - px
        dy = ysc - py
        d2 = dx * dx + dy * dy
        valid = (d2 < r2) & (pz > 0.0)
        zc = jnp.concatenate([z8, jnp.where(valid, pz, inf)], axis=0)
        dc = jnp.concatenate([d8, d2], axis=0)
        oc = jnp.concatenate([o8, jnp.where(valid, oid, 1.0e9)], axis=0)

        zrows, drows, orows = [], [], []
        for _ in range(K):
            zmin = jnp.min(zc, axis=0, keepdims=True)
            eq = zc == zmin
            omin = jnp.min(jnp.where(eq, oc, 1.0e9), axis=0, keepdims=True)
            sel = eq & (oc == omin)
            dmin = jnp.min(jnp.where(sel, dc, inf), axis=0, keepdims=True)
            dmin = jnp.where(zmin < inf, dmin, 0.0)
            zrows.append(zmin)
            drows.append(dmin)
            orows.append(omin)
            zc = jnp.where(sel, inf, zc)
        return (jnp.concatenate(zrows, axis=0),
                jnp.concatenate(drows, axis=0),
                jnp.concatenate(orows, axis=0))

    z8, d8, o8 = lax.fori_loop(0, nw, wbody, (z8, d8, o8))

    ok = z8 < inf
    dist = jnp.where(ok, d8, -1.0) / r2
    a = 1.0 - jnp.sqrt(jnp.clip(dist, 0.001, 1.0))
    a = jnp.where(ok, a, 0.0)
    wrows = []
    trans = jnp.ones((1, W), jnp.float32)
    for k in range(K):
        ak = a[k:k + 1, :]
        wrows.append(ak * trans)
        trans = trans * (1.0 - ak)
    krow = jax.lax.broadcasted_iota(jnp.int32, (K, W), 0)
    dummy = (lane * 31 + y * 7 + krow * 523) & (P - 1)
    oidi = jnp.where(ok, o8, 0.0).astype(jnp.int32)
    gidx = jnp.where(ok, oidi, dummy) + boff.astype(jnp.int32)
    w_ref[0, 0] = jnp.concatenate(wrows, axis=0)
    i_ref[0, 0] = gidx


def _rasterize(pts3D, image_size):
    Hf = jnp.asarray(image_size[0], dtype=jnp.float32)
    Wf = jnp.asarray(image_size[1], dtype=jnp.float32)
    bs = pts3D.shape[0]
    radius = 1.5 / Hf * 2.0
    r2 = radius * radius

    pts = pts3D * jnp.array([-1.0, -1.0, 1.0], dtype=pts3D.dtype)
    oid = jnp.broadcast_to(
        jnp.arange(P, dtype=jnp.float32)[None, :, None], (bs, P, 1))
    pts_sq = jnp.transpose(
        jnp.concatenate([pts, oid], axis=2), (0, 2, 1)).reshape(bs, 4, 32, 128)

    xs = 1.0 - (2.0 * jnp.arange(W, dtype=jnp.float32) + 1.0) / Wf
    ys = 1.0 - (2.0 * jnp.arange(H, dtype=jnp.float32) + 1.0) / Hf
    par = jnp.zeros((8, 128), dtype=jnp.float32)
    par = par.at[0, :].set(xs)
    par = par.at[1, :].set(r2)
    par = par.at[2, :].set(Hf)
    pars = [par.at[3, :].set(float(b * P)) for b in range(bs)]
    ys_arr = jnp.broadcast_to(ys[:, None, None], (H, 1, 128)).astype(jnp.float32)

    srt, lo, hi = pl.pallas_call(
        _sort_kernel,
        grid=(bs,),
        in_specs=[
            pl.BlockSpec((1, 4, 32, 128), lambda b: (b, 0, 0, 0)),
            pl.BlockSpec((8, 128), lambda b: (0, 0)),
        ],
        out_specs=[
            pl.BlockSpec((1, 4, 32, 128), lambda b: (b, 0, 0, 0)),
            pl.BlockSpec((1, 1, 128), lambda b: (b, 0, 0)),
            pl.BlockSpec((1, 1, 128), lambda b: (b, 0, 0)),
        ],
        out_shape=[
            jax.ShapeDtypeStruct((bs, 4, 32, 128), jnp.float32),
            jax.ShapeDtypeStruct((bs, 1, 128), jnp.int32),
            jax.ShapeDtypeStruct((bs, 1, 128), jnp.int32),
        ],
    )(pts_sq, par)

    st = jnp.transpose(srt.reshape(bs, 4, P), (0, 2, 1))
    tab = jnp.zeros((bs, PTAB, 128), jnp.float32)
    tab = tab.at[:, :P, 0:4].set(st)

    hh = H // 2
    outs = []
    for b in range(bs):
        for h in range(2):
            wgt_b, gidx_b = pl.pallas_call(
                _raster_kernel,
                grid=(hh,),
                in_specs=[
                    pl.BlockSpec((1, PTAB, 128), lambda y: (0, 0, 0)),
                    pl.BlockSpec((8, 128), lambda y: (0, 0)),
                    pl.BlockSpec((1, 1, 128), lambda y: (y, 0, 0)),
                    pl.BlockSpec(memory_space=pltpu.SMEM),
                    pl.BlockSpec(memory_space=pltpu.SMEM),
                ],
                out_specs=[
                    pl.BlockSpec((1, 1, K, W), lambda y: (0, y, 0, 0)),
                    pl.BlockSpec((1, 1, K, W), lambda y: (0, y, 0, 0)),
                ],
                out_shape=[
                    jax.ShapeDtypeStruct((1, hh, K, W), jnp.float32),
                    jax.ShapeDtypeStruct((1, hh, K, W), jnp.int32),
                ],
            )(tab[b:b + 1], pars[b], ys_arr[h * hh:(h + 1) * hh],
              lo[b:b + 1, :, h * hh:(h + 1) * hh],
              hi[b:b + 1, :, h * hh:(h + 1) * hh])
            outs.append((wgt_b, gidx_b))
    return outs


def _make_sc_composite(npix):
    nslot = npix * K
    info = plsc.get_sparse_core_info()
    nc, ns = info.num_cores, info.num_subcores
    nw = nc * ns
    slots_per_w = nslot // nw
    nchunk = slots_per_w // _CHUNK
    mesh = plsc.VectorSubcoreMesh(core_axis_name="c", subcore_axis_name="s")

    @functools.partial(
        pl.kernel,
        mesh=mesh,
        out_type=jax.ShapeDtypeStruct((npix, C), jnp.float32),
        scratch_types=[
            pltpu.VMEM((_CHUNK,), jnp.int32),
            pltpu.VMEM((_CHUNK,), jnp.int32),
            pltpu.VMEM((_CHUNK,), jnp.float32),
            pltpu.VMEM((_CHUNK,), jnp.float32),
            pltpu.VMEM((_CHUNK, 2 * C), jnp.float32),
            pltpu.VMEM((_CHUNK, 2 * C), jnp.float32),
            pltpu.VMEM((_CHUNK // K, C), jnp.float32),
            pltpu.SemaphoreType.DMA,
            pltpu.SemaphoreType.DMA,
        ],
    )
    def composite(idx_hbm, w_hbm, feats_hbm, out_hbm, idx_a, idx_b,
                  w_a, w_b, rows_a, rows_b, acc_v, sem_a, sem_b):
        wid = lax.axis_index("s") * nc + lax.axis_index("c")
        base = wid * slots_per_w

        def compute(rows_v, w_v, off):
            for pair in range(_CHUNK // 16):
                r0 = pair * 16
                wblk = w_v[pl.ds(r0, 16)]
                for sub in range(2):
                    px = pair * 2 + sub
                    for cs in range(C // 16):
                        acc = jnp.zeros((16,), jnp.float32)
                        for k in range(K):
                            wv = _lane_bcast(wblk, sub * K + k)
                            acc = acc + wv * rows_v[
                                r0 + sub * K + k, pl.ds(cs * 16, 16)]
                        acc_v[px, pl.ds(cs * 16, 16)] = acc
            pltpu.sync_copy(
                acc_v,
                out_hbm.at[pl.ds(pl.multiple_of(off // K, _CHUNK // K),
                                 _CHUNK // K)])

        def load(buf_i, buf_w, off):
            pltpu.sync_copy(idx_hbm.at[pl.ds(off, _CHUNK)], buf_i)
            pltpu.sync_copy(w_hbm.at[pl.ds(off, _CHUNK)], buf_w)

        npair = nchunk // 2
        off0 = pl.multiple_of(base, _CHUNK)
        load(idx_a, w_a, off0)
        gather_a = pltpu.async_copy(feats_hbm.at[idx_a], rows_a, sem_a)

        def pair_body(g, _):
            off_e = pl.multiple_of(base + (2 * g) * _CHUNK, _CHUNK)
            off_o = pl.multiple_of(off_e + _CHUNK, _CHUNK)
            off_n = pl.multiple_of(off_o + _CHUNK, _CHUNK)
            load(idx_b, w_b, off_o)
            cp_b = pltpu.async_copy(feats_hbm.at[idx_b], rows_b, sem_b)
            pltpu.make_async_copy(feats_hbm.at[idx_a], rows_a, sem_a).wait()
            compute(rows_a, w_a, off_e)

            @pl.when(g < npair - 1)
            def _():
                load(idx_a, w_a, off_n)
                pltpu.async_copy(feats_hbm.at[idx_a], rows_a, sem_a)

            cp_b.wait()
            compute(rows_b, w_b, off_o)
            return ()

        lax.fori_loop(0, npair, pair_body, ())
        del gather_a

    return composite


def kernel(pts3D, src, image_size):
    bs = pts3D.shape[0]
    per_batch = _rasterize(pts3D, image_size)
    feats = jnp.zeros((bs * P, 2 * C), jnp.float32)
    feats = feats.at[:, :C].set(jnp.transpose(src, (0, 2, 1)).reshape(bs * P, C))
    npix_b = (H // 2) * W
    comp = _make_sc_composite(npix_b)
    outs = []
    for wgt_b, gidx_b in per_batch:
        idx_flat = jnp.transpose(gidx_b, (0, 1, 3, 2)).reshape(npix_b * K)
        w_flat = jnp.transpose(wgt_b, (0, 1, 3, 2)).reshape(npix_b * K)
        outs.append(comp(idx_flat, w_flat, feats))
    out = jnp.stack(outs).reshape(bs, H, W, C)
    return jnp.transpose(out, (0, 3, 1, 2))
